# baseline (speedup 1.0000x reference)
import functools

import jax
import jax.numpy as jnp
from jax.experimental import pallas as pl
from jax.experimental.pallas import tpu as pltpu

M_POINTS = 1024
NUM_VIEW = 300
GRASPNESS_THRESHOLD = 0.1

LANES = 128
ROWS = 160
NPAD = ROWS * LANES
NB = 2048


def _fps_body(x_ref, y_ref, z_ref, o0_ref, o1_ref, gr_ref,
              sel_ref, gx_ref, gy_ref, gz_ref, *dist_refs, n_points, batch):
    rows = jax.lax.broadcasted_iota(jnp.int32, (ROWS, LANES), 0)
    cols = jax.lax.broadcasted_iota(jnp.int32, (ROWS, LANES), 1)
    nidx = rows * LANES + cols
    valid = nidx < n_points
    big = jnp.int32(2 ** 30)
    d_valid = jnp.where(valid, jnp.float32(1e10), -jnp.inf)

    state0 = []
    for b in range(batch):
        mask = ((o1_ref[b] > o0_ref[b])
                & (gr_ref[b] > jnp.float32(GRASPNESS_THRESHOLD)) & valid)
        count = jnp.sum(mask.astype(jnp.int32))
        d_mask = jnp.where(mask, jnp.float32(1e10), -jnp.inf)
        dist0 = jnp.where(count >= M_POINTS, d_mask, d_valid)
        far0 = jnp.min(jnp.where(dist0 > 0, nidx, big))
        dist_refs[b][...] = dist0
        hit0 = nidx == far0
        xf0 = jnp.sum(jnp.where(hit0, x_ref[b], 0.0))
        yf0 = jnp.sum(jnp.where(hit0, y_ref[b], 0.0))
        zf0 = jnp.sum(jnp.where(hit0, z_ref[b], 0.0))
        state0.append((jnp.broadcast_to(far0, (1, 128)),
                       jnp.broadcast_to(xf0, (1, 128)),
                       jnp.broadcast_to(yf0, (1, 128)),
                       jnp.broadcast_to(zf0, (1, 128))))
    far_all0 = jnp.concatenate([s[0] for s in state0], axis=0)
    xf_all0 = jnp.concatenate([s[1] for s in state0], axis=0)
    yf_all0 = jnp.concatenate([s[2] for s in state0], axis=0)
    zf_all0 = jnp.concatenate([s[3] for s in state0], axis=0)

    midx = (jax.lax.broadcasted_iota(jnp.int32, (8, 128), 0) * 128
            + jax.lax.broadcasted_iota(jnp.int32, (8, 128), 1))
    iota8 = midx
    n_chunks = ROWS // 8
    neg_inf8 = jnp.full((8, 128), -jnp.inf, jnp.float32)
    zero_i8 = jnp.zeros((8, 128), jnp.int32)
    zero_f8 = jnp.zeros((8, 128), jnp.float32)

    def body(i, state):
        far_all, xf_all, yf_all, zf_all = state
        wsel = midx == i
        xfv, yfv, zfv = [], [], []
        for b in range(batch):
            farv = jnp.broadcast_to(far_all[b:b + 1, :], (8, 128))
            xfb = jnp.broadcast_to(xf_all[b:b + 1, :], (8, 128))
            yfb = jnp.broadcast_to(yf_all[b:b + 1, :], (8, 128))
            zfb = jnp.broadcast_to(zf_all[b:b + 1, :], (8, 128))
            sel_ref[b] = jnp.where(wsel, farv, sel_ref[b])
            gx_ref[b] = jnp.where(wsel, xfb, gx_ref[b])
            gy_ref[b] = jnp.where(wsel, yfb, gy_ref[b])
            gz_ref[b] = jnp.where(wsel, zfb, gz_ref[b])
            xfv.append(xfb)
            yfv.append(yfb)
            zfv.append(zfb)

        runval = [neg_inf8] * batch
        runidx = [zero_i8] * batch
        runx = [zero_f8] * batch
        runy = [zero_f8] * batch
        runz = [zero_f8] * batch
        for r in range(n_chunks):
            sl = pl.ds(r * 8, 8)
            nidx_r = iota8 + jnp.int32(r * 1024)
            for b in range(batch):
                xc = x_ref[b, sl, :]
                yc = y_ref[b, sl, :]
                zc = z_ref[b, sl, :]
                dx = xc - xfv[b]
                dy = yc - yfv[b]
                dz = zc - zfv[b]
                d = dx * dx + dy * dy + dz * dz
                dist = jnp.minimum(dist_refs[b][sl, :], d)
                dist_refs[b][sl, :] = dist
                gtr = dist > runval[b]
                runval[b] = jnp.where(gtr, dist, runval[b])
                runidx[b] = jnp.where(gtr, nidx_r, runidx[b])
                runx[b] = jnp.where(gtr, xc, runx[b])
                runy[b] = jnp.where(gtr, yc, runy[b])
                runz[b] = jnp.where(gtr, zc, runz[b])

        vrow = jnp.concatenate(
            [jnp.max(runval[b], axis=0, keepdims=True) for b in range(batch)],
            axis=0)
        mx_all = jnp.broadcast_to(jnp.max(vrow, axis=1, keepdims=True),
                                  (batch, 128))
        eligs, cands = [], []
        for b in range(batch):
            mxb = jnp.broadcast_to(mx_all[b:b + 1, :], (8, 128))
            elig = runval[b] == mxb
            eligs.append(elig)
            cands.append(jnp.min(
                jnp.where(elig, runidx[b], big), axis=0, keepdims=True))
        crow = jnp.concatenate(cands, axis=0)
        far_n = jnp.broadcast_to(jnp.min(crow, axis=1, keepdims=True),
                                 (batch, 128))
        xs, ys, zs = [], [], []
        for b in range(batch):
            farb = jnp.broadcast_to(far_n[b:b + 1, :], (8, 128))
            hit = eligs[b] & (runidx[b] == farb)
            xs.append(jnp.sum(jnp.where(hit, runx[b], 0.0), axis=0,
                              keepdims=True))
            ys.append(jnp.sum(jnp.where(hit, runy[b], 0.0), axis=0,
                              keepdims=True))
            zs.append(jnp.sum(jnp.where(hit, runz[b], 0.0), axis=0,
                              keepdims=True))
        xf_n = jnp.broadcast_to(
            jnp.sum(jnp.concatenate(xs, axis=0), axis=1, keepdims=True),
            (batch, 128))
        yf_n = jnp.broadcast_to(
            jnp.sum(jnp.concatenate(ys, axis=0), axis=1, keepdims=True),
            (batch, 128))
        zf_n = jnp.broadcast_to(
            jnp.sum(jnp.concatenate(zs, axis=0), axis=1, keepdims=True),
            (batch, 128))
        return (far_n, xf_n, yf_n, zf_n)

    jax.lax.fori_loop(0, M_POINTS, body,
                      (far_all0, xf_all0, yf_all0, zf_all0))


def _gather_body(sel_ref, sf_ref, w_ref, feat_ref, view_ref, acc_ref,
                 *, n_points, n_blocks):
    n = pl.program_id(1)

    @pl.when(n == 0)
    def _():
        acc_ref[...] = jnp.zeros_like(acc_ref)

    col0 = n * NB
    lane_cols = col0 + jax.lax.broadcasted_iota(jnp.int32, (1, NB), 1)
    sfb = jnp.where(lane_cols < n_points, sf_ref[0], 0.0).astype(jnp.bfloat16)
    sub_cols = col0 + jax.lax.broadcasted_iota(jnp.int32, (NB, 1), 0)
    onehot = jnp.where(sub_cols == sel_ref[0], jnp.float32(1), jnp.float32(0)
                       ).astype(jnp.bfloat16)
    acc_ref[...] += jax.lax.dot_general(
        sfb, onehot, (((1,), (0,)), ((), ())),
        preferred_element_type=jnp.float32)

    @pl.when(n == n_blocks - 1)
    def _():
        feat = acc_ref[...]
        feat_ref[0] = feat
        view_ref[0] = jax.lax.dot_general(
            w_ref[...], feat, (((1,), (0,)), ((), ())),
            preferred_element_type=jnp.float32)


def kernel(point_clouds, seed_features, objectness_score, graspness_score, W_view):
    B, N, _ = point_clouds.shape
    C = seed_features.shape[1]

    def prep(a):
        return jnp.pad(a, ((0, 0), (0, NPAD - N))).reshape(B, ROWS, LANES)

    x = prep(point_clouds[:, :, 0])
    y = prep(point_clouds[:, :, 1])
    z = prep(point_clouds[:, :, 2])
    o0 = prep(objectness_score[:, 0, :])
    o1 = prep(objectness_score[:, 1, :])
    gr = prep(graspness_score[:, 0, :])

    GB = 2
    blk = pl.BlockSpec((GB, ROWS, LANES), lambda i: (i, 0, 0))
    oblk = pl.BlockSpec((GB, 8, 128), lambda i: (i, 0, 0))
    sel8, gx, gy, gz = pl.pallas_call(
        functools.partial(_fps_body, n_points=N, batch=GB),
        grid=(B // GB,),
        in_specs=[blk] * 6,
        out_specs=[oblk, oblk, oblk, oblk],
        out_shape=[
            jax.ShapeDtypeStruct((B, 8, 128), jnp.int32),
            jax.ShapeDtypeStruct((B, 8, 128), jnp.float32),
            jax.ShapeDtypeStruct((B, 8, 128), jnp.float32),
            jax.ShapeDtypeStruct((B, 8, 128), jnp.float32),
        ],
        scratch_shapes=[pltpu.VMEM((ROWS, LANES), jnp.float32)
                        for _ in range(GB)],
        compiler_params=pltpu.CompilerParams(
            dimension_semantics=("parallel",)),
    )(x, y, z, o0, o1, gr)

    sel = sel8.reshape(B, M_POINTS)
    xyz_g = jnp.stack(
        [gx.reshape(B, M_POINTS), gy.reshape(B, M_POINTS),
         gz.reshape(B, M_POINTS)], axis=-1)

    n_blocks = pl.cdiv(N, NB)
    feat_g, view_score = pl.pallas_call(
        functools.partial(_gather_body, n_points=N, n_blocks=n_blocks),
        grid=(B, n_blocks),
        in_specs=[
            pl.BlockSpec((1, 1, M_POINTS), lambda b, n: (b, 0, 0)),
            pl.BlockSpec((1, C, NB), lambda b, n: (b, 0, n)),
            pl.BlockSpec((NUM_VIEW, C), lambda b, n: (0, 0)),
        ],
        out_specs=[
            pl.BlockSpec((1, C, M_POINTS), lambda b, n: (b, 0, 0)),
            pl.BlockSpec((1, NUM_VIEW, M_POINTS), lambda b, n: (b, 0, 0)),
        ],
        out_shape=[
            jax.ShapeDtypeStruct((B, C, M_POINTS), jnp.float32),
            jax.ShapeDtypeStruct((B, NUM_VIEW, M_POINTS), jnp.float32),
        ],
        scratch_shapes=[pltpu.VMEM((C, M_POINTS), jnp.float32)],
        compiler_params=pltpu.CompilerParams(
            dimension_semantics=("parallel", "arbitrary")),
    )(sel.reshape(B, 1, M_POINTS), seed_features, W_view)

    return view_score, xyz_g, feat_g


# speedup vs baseline: 1.4371x; 1.4371x over previous
import functools

import jax
import jax.numpy as jnp
from jax.experimental import pallas as pl
from jax.experimental.pallas import tpu as pltpu

M_POINTS = 1024
NUM_VIEW = 300
GRASPNESS_THRESHOLD = 0.1

LANES = 128
ROWS = 160
NPAD = ROWS * LANES
NB = 2048


def _fps_body(x_ref, y_ref, z_ref, o0_ref, o1_ref, gr_ref,
              sel_ref, gx_ref, gy_ref, gz_ref, *dist_refs, n_points, batch):
    rows = jax.lax.broadcasted_iota(jnp.int32, (ROWS, LANES), 0)
    cols = jax.lax.broadcasted_iota(jnp.int32, (ROWS, LANES), 1)
    nidx = rows * LANES + cols
    valid = nidx < n_points
    big = jnp.int32(2 ** 30)
    d_valid = jnp.where(valid, jnp.float32(1e10), -jnp.inf)

    state0 = []
    for b in range(batch):
        mask = ((o1_ref[b] > o0_ref[b])
                & (gr_ref[b] > jnp.float32(GRASPNESS_THRESHOLD)) & valid)
        count = jnp.sum(mask.astype(jnp.int32))
        d_mask = jnp.where(mask, jnp.float32(1e10), -jnp.inf)
        dist0 = jnp.where(count >= M_POINTS, d_mask, d_valid)
        far0 = jnp.min(jnp.where(dist0 > 0, nidx, big))
        dist_refs[b][...] = dist0
        hit0 = nidx == far0
        xf0 = jnp.sum(jnp.where(hit0, x_ref[b], 0.0))
        yf0 = jnp.sum(jnp.where(hit0, y_ref[b], 0.0))
        zf0 = jnp.sum(jnp.where(hit0, z_ref[b], 0.0))
        state0.append((jnp.broadcast_to(far0, (1, 128)),
                       jnp.broadcast_to(xf0, (1, 128)),
                       jnp.broadcast_to(yf0, (1, 128)),
                       jnp.broadcast_to(zf0, (1, 128))))
    far_all0 = jnp.concatenate([s[0] for s in state0], axis=0)
    xf_all0 = jnp.concatenate([s[1] for s in state0], axis=0)
    yf_all0 = jnp.concatenate([s[2] for s in state0], axis=0)
    zf_all0 = jnp.concatenate([s[3] for s in state0], axis=0)

    midx = (jax.lax.broadcasted_iota(jnp.int32, (8, 128), 0) * 128
            + jax.lax.broadcasted_iota(jnp.int32, (8, 128), 1))
    iota8 = midx
    n_chunks = ROWS // 8
    neg_inf8 = jnp.full((8, 128), -jnp.inf, jnp.float32)
    zero_i8 = jnp.zeros((8, 128), jnp.int32)
    zero_f8 = jnp.zeros((8, 128), jnp.float32)

    def body(i, state):
        far_all, xf_all, yf_all, zf_all = state
        wsel = midx == i
        xfv, yfv, zfv = [], [], []
        for b in range(batch):
            farv = jnp.broadcast_to(far_all[b:b + 1, :], (8, 128))
            xfb = jnp.broadcast_to(xf_all[b:b + 1, :], (8, 128))
            yfb = jnp.broadcast_to(yf_all[b:b + 1, :], (8, 128))
            zfb = jnp.broadcast_to(zf_all[b:b + 1, :], (8, 128))
            sel_ref[b] = jnp.where(wsel, farv, sel_ref[b])
            gx_ref[b] = jnp.where(wsel, xfb, gx_ref[b])
            gy_ref[b] = jnp.where(wsel, yfb, gy_ref[b])
            gz_ref[b] = jnp.where(wsel, zfb, gz_ref[b])
            xfv.append(xfb)
            yfv.append(yfb)
            zfv.append(zfb)

        runval = [neg_inf8] * batch
        runidx = [zero_i8] * batch
        runx = [zero_f8] * batch
        runy = [zero_f8] * batch
        runz = [zero_f8] * batch
        for b in range(batch):
            for r in range(n_chunks):
                sl = pl.ds(r * 8, 8)
                nidx_r = iota8 + jnp.int32(r * 1024)
                xc = x_ref[b, sl, :]
                yc = y_ref[b, sl, :]
                zc = z_ref[b, sl, :]
                dx = xc - xfv[b]
                dy = yc - yfv[b]
                dz = zc - zfv[b]
                d = dx * dx + dy * dy + dz * dz
                dist = jnp.minimum(dist_refs[b][sl, :], d)
                dist_refs[b][sl, :] = dist
                gtr = dist > runval[b]
                runval[b] = jnp.where(gtr, dist, runval[b])
                runidx[b] = jnp.where(gtr, nidx_r, runidx[b])
                runx[b] = jnp.where(gtr, xc, runx[b])
                runy[b] = jnp.where(gtr, yc, runy[b])
                runz[b] = jnp.where(gtr, zc, runz[b])

        vrow = jnp.concatenate(
            [jnp.max(runval[b], axis=0, keepdims=True) for b in range(batch)],
            axis=0)
        mx_all = jnp.broadcast_to(jnp.max(vrow, axis=1, keepdims=True),
                                  (batch, 128))
        eligs, cands = [], []
        for b in range(batch):
            mxb = jnp.broadcast_to(mx_all[b:b + 1, :], (8, 128))
            elig = runval[b] == mxb
            eligs.append(elig)
            cands.append(jnp.min(
                jnp.where(elig, runidx[b], big), axis=0, keepdims=True))
        crow = jnp.concatenate(cands, axis=0)
        far_n = jnp.broadcast_to(jnp.min(crow, axis=1, keepdims=True),
                                 (batch, 128))
        xs, ys, zs = [], [], []
        for b in range(batch):
            farb = jnp.broadcast_to(far_n[b:b + 1, :], (8, 128))
            hit = eligs[b] & (runidx[b] == farb)
            xs.append(jnp.sum(jnp.where(hit, runx[b], 0.0), axis=0,
                              keepdims=True))
            ys.append(jnp.sum(jnp.where(hit, runy[b], 0.0), axis=0,
                              keepdims=True))
            zs.append(jnp.sum(jnp.where(hit, runz[b], 0.0), axis=0,
                              keepdims=True))
        xf_n = jnp.broadcast_to(
            jnp.sum(jnp.concatenate(xs, axis=0), axis=1, keepdims=True),
            (batch, 128))
        yf_n = jnp.broadcast_to(
            jnp.sum(jnp.concatenate(ys, axis=0), axis=1, keepdims=True),
            (batch, 128))
        zf_n = jnp.broadcast_to(
            jnp.sum(jnp.concatenate(zs, axis=0), axis=1, keepdims=True),
            (batch, 128))
        return (far_n, xf_n, yf_n, zf_n)

    jax.lax.fori_loop(0, M_POINTS, body,
                      (far_all0, xf_all0, yf_all0, zf_all0))


def _gather_body(sel_ref, sf_ref, w_ref, feat_ref, view_ref, acc_ref,
                 *, n_points, n_blocks):
    n = pl.program_id(1)

    @pl.when(n == 0)
    def _():
        acc_ref[...] = jnp.zeros_like(acc_ref)

    col0 = n * NB
    lane_cols = col0 + jax.lax.broadcasted_iota(jnp.int32, (1, NB), 1)
    sfb = jnp.where(lane_cols < n_points, sf_ref[0], 0.0).astype(jnp.bfloat16)
    sub_cols = col0 + jax.lax.broadcasted_iota(jnp.int32, (NB, 1), 0)
    onehot = jnp.where(sub_cols == sel_ref[0], jnp.float32(1), jnp.float32(0)
                       ).astype(jnp.bfloat16)
    acc_ref[...] += jax.lax.dot_general(
        sfb, onehot, (((1,), (0,)), ((), ())),
        preferred_element_type=jnp.float32)

    @pl.when(n == n_blocks - 1)
    def _():
        feat = acc_ref[...]
        feat_ref[0] = feat
        view_ref[0] = jax.lax.dot_general(
            w_ref[...], feat, (((1,), (0,)), ((), ())),
            preferred_element_type=jnp.float32)


def kernel(point_clouds, seed_features, objectness_score, graspness_score, W_view):
    B, N, _ = point_clouds.shape
    C = seed_features.shape[1]

    def prep(a):
        return jnp.pad(a, ((0, 0), (0, NPAD - N))).reshape(B, ROWS, LANES)

    x = prep(point_clouds[:, :, 0])
    y = prep(point_clouds[:, :, 1])
    z = prep(point_clouds[:, :, 2])
    o0 = prep(objectness_score[:, 0, :])
    o1 = prep(objectness_score[:, 1, :])
    gr = prep(graspness_score[:, 0, :])

    blk = pl.BlockSpec((B, ROWS, LANES), lambda: (0, 0, 0))
    oblk = pl.BlockSpec((B, 8, 128), lambda: (0, 0, 0))
    sel8, gx, gy, gz = pl.pallas_call(
        functools.partial(_fps_body, n_points=N, batch=B),
        in_specs=[blk] * 6,
        out_specs=[oblk, oblk, oblk, oblk],
        out_shape=[
            jax.ShapeDtypeStruct((B, 8, 128), jnp.int32),
            jax.ShapeDtypeStruct((B, 8, 128), jnp.float32),
            jax.ShapeDtypeStruct((B, 8, 128), jnp.float32),
            jax.ShapeDtypeStruct((B, 8, 128), jnp.float32),
        ],
        scratch_shapes=[pltpu.VMEM((ROWS, LANES), jnp.float32)
                        for _ in range(B)],
    )(x, y, z, o0, o1, gr)

    sel = sel8.reshape(B, M_POINTS)
    xyz_g = jnp.stack(
        [gx.reshape(B, M_POINTS), gy.reshape(B, M_POINTS),
         gz.reshape(B, M_POINTS)], axis=-1)

    n_blocks = pl.cdiv(N, NB)
    feat_g, view_score = pl.pallas_call(
        functools.partial(_gather_body, n_points=N, n_blocks=n_blocks),
        grid=(B, n_blocks),
        in_specs=[
            pl.BlockSpec((1, 1, M_POINTS), lambda b, n: (b, 0, 0)),
            pl.BlockSpec((1, C, NB), lambda b, n: (b, 0, n)),
            pl.BlockSpec((NUM_VIEW, C), lambda b, n: (0, 0)),
        ],
        out_specs=[
            pl.BlockSpec((1, C, M_POINTS), lambda b, n: (b, 0, 0)),
            pl.BlockSpec((1, NUM_VIEW, M_POINTS), lambda b, n: (b, 0, 0)),
        ],
        out_shape=[
            jax.ShapeDtypeStruct((B, C, M_POINTS), jnp.float32),
            jax.ShapeDtypeStruct((B, NUM_VIEW, M_POINTS), jnp.float32),
        ],
        scratch_shapes=[pltpu.VMEM((C, M_POINTS), jnp.float32)],
        compiler_params=pltpu.CompilerParams(
            dimension_semantics=("parallel", "arbitrary")),
    )(sel.reshape(B, 1, M_POINTS), seed_features, W_view)

    return view_score, xyz_g, feat_g


# speedup vs baseline: 1.4442x; 1.0050x over previous
import functools

import jax
import jax.numpy as jnp
from jax.experimental import pallas as pl
from jax.experimental.pallas import tpu as pltpu

M_POINTS = 1024
NUM_VIEW = 300
GRASPNESS_THRESHOLD = 0.1

LANES = 128
ROWS = 160
NPAD = ROWS * LANES
NB = 2048


def _fps_body(x_ref, y_ref, z_ref, o0_ref, o1_ref, gr_ref,
              sel_ref, gx_ref, gy_ref, gz_ref, *dist_refs, n_points, batch):
    rows = jax.lax.broadcasted_iota(jnp.int32, (ROWS, LANES), 0)
    cols = jax.lax.broadcasted_iota(jnp.int32, (ROWS, LANES), 1)
    nidx = rows * LANES + cols
    valid = nidx < n_points
    big = jnp.int32(2 ** 30)
    d_valid = jnp.where(valid, jnp.float32(1e10), -jnp.inf)

    state0 = []
    for b in range(batch):
        mask = ((o1_ref[b] > o0_ref[b])
                & (gr_ref[b] > jnp.float32(GRASPNESS_THRESHOLD)) & valid)
        count = jnp.sum(mask.astype(jnp.int32))
        d_mask = jnp.where(mask, jnp.float32(1e10), -jnp.inf)
        dist0 = jnp.where(count >= M_POINTS, d_mask, d_valid)
        far0 = jnp.min(jnp.where(dist0 > 0, nidx, big))
        dist_refs[b][...] = dist0
        hit0 = nidx == far0
        xf0 = jnp.sum(jnp.where(hit0, x_ref[b], 0.0))
        yf0 = jnp.sum(jnp.where(hit0, y_ref[b], 0.0))
        zf0 = jnp.sum(jnp.where(hit0, z_ref[b], 0.0))
        state0.append((jnp.broadcast_to(far0, (1, 128)),
                       jnp.broadcast_to(xf0, (1, 128)),
                       jnp.broadcast_to(yf0, (1, 128)),
                       jnp.broadcast_to(zf0, (1, 128))))
    far_all0 = jnp.concatenate([s[0] for s in state0], axis=0)
    xf_all0 = jnp.concatenate([s[1] for s in state0], axis=0)
    yf_all0 = jnp.concatenate([s[2] for s in state0], axis=0)
    zf_all0 = jnp.concatenate([s[3] for s in state0], axis=0)

    midx = (jax.lax.broadcasted_iota(jnp.int32, (8, 128), 0) * 128
            + jax.lax.broadcasted_iota(jnp.int32, (8, 128), 1))
    iota8 = midx
    n_chunks = ROWS // 8
    neg_inf8 = jnp.full((8, 128), -jnp.inf, jnp.float32)
    zero_i8 = jnp.zeros((8, 128), jnp.int32)
    zero_f8 = jnp.zeros((8, 128), jnp.float32)

    def body(i, state):
        far_all, xf_all, yf_all, zf_all = state
        wsel = midx == i
        xfv, yfv, zfv = [], [], []
        for b in range(batch):
            farv = jnp.broadcast_to(far_all[b:b + 1, :], (8, 128))
            xfb = jnp.broadcast_to(xf_all[b:b + 1, :], (8, 128))
            yfb = jnp.broadcast_to(yf_all[b:b + 1, :], (8, 128))
            zfb = jnp.broadcast_to(zf_all[b:b + 1, :], (8, 128))
            sel_ref[b] = jnp.where(wsel, farv, sel_ref[b])
            gx_ref[b] = jnp.where(wsel, xfb, gx_ref[b])
            gy_ref[b] = jnp.where(wsel, yfb, gy_ref[b])
            gz_ref[b] = jnp.where(wsel, zfb, gz_ref[b])
            xfv.append(xfb)
            yfv.append(yfb)
            zfv.append(zfb)

        runval = [neg_inf8] * batch
        runidx = [zero_i8] * batch
        runx = [zero_f8] * batch
        runy = [zero_f8] * batch
        runz = [zero_f8] * batch
        for b in range(batch):
            for r in range(n_chunks):
                sl = pl.ds(r * 8, 8)
                nidx_r = iota8 + jnp.int32(r * 1024)
                xc = x_ref[b, sl, :]
                yc = y_ref[b, sl, :]
                zc = z_ref[b, sl, :]
                dx = xc - xfv[b]
                dy = yc - yfv[b]
                dz = zc - zfv[b]
                d = dx * dx + dy * dy + dz * dz
                dist = jnp.minimum(dist_refs[b][sl, :], d)
                dist_refs[b][sl, :] = dist
                gtr = dist > runval[b]
                runval[b] = jnp.where(gtr, dist, runval[b])
                runidx[b] = jnp.where(gtr, nidx_r, runidx[b])
                runx[b] = jnp.where(gtr, xc, runx[b])
                runy[b] = jnp.where(gtr, yc, runy[b])
                runz[b] = jnp.where(gtr, zc, runz[b])

        vrow = jnp.concatenate(
            [jnp.max(runval[b], axis=0, keepdims=True) for b in range(batch)],
            axis=0)
        mx_all = jnp.broadcast_to(jnp.max(vrow, axis=1, keepdims=True),
                                  (batch, 128))
        eligs, cands = [], []
        for b in range(batch):
            mxb = jnp.broadcast_to(mx_all[b:b + 1, :], (8, 128))
            elig = runval[b] == mxb
            eligs.append(elig)
            cands.append(jnp.min(
                jnp.where(elig, runidx[b], big), axis=0, keepdims=True))
        crow = jnp.concatenate(cands, axis=0)
        far_n = jnp.broadcast_to(jnp.min(crow, axis=1, keepdims=True),
                                 (batch, 128))
        xs, ys, zs = [], [], []
        for b in range(batch):
            farb = jnp.broadcast_to(far_n[b:b + 1, :], (8, 128))
            hit = eligs[b] & (runidx[b] == farb)
            xs.append(jnp.sum(jnp.where(hit, runx[b], 0.0), axis=0,
                              keepdims=True))
            ys.append(jnp.sum(jnp.where(hit, runy[b], 0.0), axis=0,
                              keepdims=True))
            zs.append(jnp.sum(jnp.where(hit, runz[b], 0.0), axis=0,
                              keepdims=True))
        xf_n = jnp.broadcast_to(
            jnp.sum(jnp.concatenate(xs, axis=0), axis=1, keepdims=True),
            (batch, 128))
        yf_n = jnp.broadcast_to(
            jnp.sum(jnp.concatenate(ys, axis=0), axis=1, keepdims=True),
            (batch, 128))
        zf_n = jnp.broadcast_to(
            jnp.sum(jnp.concatenate(zs, axis=0), axis=1, keepdims=True),
            (batch, 128))
        return (far_n, xf_n, yf_n, zf_n)

    jax.lax.fori_loop(0, M_POINTS, body,
                      (far_all0, xf_all0, yf_all0, zf_all0), unroll=2)


def _gather_body(sel_ref, sf_ref, w_ref, feat_ref, view_ref, acc_ref,
                 *, n_points, n_blocks):
    n = pl.program_id(1)

    @pl.when(n == 0)
    def _():
        acc_ref[...] = jnp.zeros_like(acc_ref)

    col0 = n * NB
    lane_cols = col0 + jax.lax.broadcasted_iota(jnp.int32, (1, NB), 1)
    sfb = jnp.where(lane_cols < n_points, sf_ref[0], 0.0).astype(jnp.bfloat16)
    sub_cols = col0 + jax.lax.broadcasted_iota(jnp.int32, (NB, 1), 0)
    onehot = jnp.where(sub_cols == sel_ref[0], jnp.float32(1), jnp.float32(0)
                       ).astype(jnp.bfloat16)
    acc_ref[...] += jax.lax.dot_general(
        sfb, onehot, (((1,), (0,)), ((), ())),
        preferred_element_type=jnp.float32)

    @pl.when(n == n_blocks - 1)
    def _():
        feat = acc_ref[...]
        feat_ref[0] = feat
        view_ref[0] = jax.lax.dot_general(
            w_ref[...], feat, (((1,), (0,)), ((), ())),
            preferred_element_type=jnp.float32)


def kernel(point_clouds, seed_features, objectness_score, graspness_score, W_view):
    B, N, _ = point_clouds.shape
    C = seed_features.shape[1]

    def prep(a):
        return jnp.pad(a, ((0, 0), (0, NPAD - N))).reshape(B, ROWS, LANES)

    x = prep(point_clouds[:, :, 0])
    y = prep(point_clouds[:, :, 1])
    z = prep(point_clouds[:, :, 2])
    o0 = prep(objectness_score[:, 0, :])
    o1 = prep(objectness_score[:, 1, :])
    gr = prep(graspness_score[:, 0, :])

    blk = pl.BlockSpec((B, ROWS, LANES), lambda: (0, 0, 0))
    oblk = pl.BlockSpec((B, 8, 128), lambda: (0, 0, 0))
    sel8, gx, gy, gz = pl.pallas_call(
        functools.partial(_fps_body, n_points=N, batch=B),
        in_specs=[blk] * 6,
        out_specs=[oblk, oblk, oblk, oblk],
        out_shape=[
            jax.ShapeDtypeStruct((B, 8, 128), jnp.int32),
            jax.ShapeDtypeStruct((B, 8, 128), jnp.float32),
            jax.ShapeDtypeStruct((B, 8, 128), jnp.float32),
            jax.ShapeDtypeStruct((B, 8, 128), jnp.float32),
        ],
        scratch_shapes=[pltpu.VMEM((ROWS, LANES), jnp.float32)
                        for _ in range(B)],
    )(x, y, z, o0, o1, gr)

    sel = sel8.reshape(B, M_POINTS)
    xyz_g = jnp.stack(
        [gx.reshape(B, M_POINTS), gy.reshape(B, M_POINTS),
         gz.reshape(B, M_POINTS)], axis=-1)

    n_blocks = pl.cdiv(N, NB)
    feat_g, view_score = pl.pallas_call(
        functools.partial(_gather_body, n_points=N, n_blocks=n_blocks),
        grid=(B, n_blocks),
        in_specs=[
            pl.BlockSpec((1, 1, M_POINTS), lambda b, n: (b, 0, 0)),
            pl.BlockSpec((1, C, NB), lambda b, n: (b, 0, n)),
            pl.BlockSpec((NUM_VIEW, C), lambda b, n: (0, 0)),
        ],
        out_specs=[
            pl.BlockSpec((1, C, M_POINTS), lambda b, n: (b, 0, 0)),
            pl.BlockSpec((1, NUM_VIEW, M_POINTS), lambda b, n: (b, 0, 0)),
        ],
        out_shape=[
            jax.ShapeDtypeStruct((B, C, M_POINTS), jnp.float32),
            jax.ShapeDtypeStruct((B, NUM_VIEW, M_POINTS), jnp.float32),
        ],
        scratch_shapes=[pltpu.VMEM((C, M_POINTS), jnp.float32)],
        compiler_params=pltpu.CompilerParams(
            dimension_semantics=("parallel", "arbitrary")),
    )(sel.reshape(B, 1, M_POINTS), seed_features, W_view)

    return view_score, xyz_g, feat_g
